# Initial kernel scaffold; baseline (speedup 1.0000x reference)
import jax
import jax.numpy as jnp
from jax.experimental import pallas as pl


def kernel(x, edge_index, W_init, b_init, W_ord, b_ord, ln_g, ln_b, Wp, bp):
    raise NotImplementedError("write your pallas kernel here")


# speedup vs baseline: 14.5700x; 14.5700x over previous
import functools

import jax
import jax.numpy as jnp
from jax import lax
from jax.experimental import pallas as pl
from jax.experimental.pallas import tpu as pltpu, tpu_sc as plsc

N = 10000
E = 320000
H = 128

NW = 32
K = 80
EPW = E // NW
NBLK = EPW // K
NPAD = 10240
RPS = NPAD // 16
BR = 1024
GRID = NPAD // BR

_mesh = plsc.VectorSubcoreMesh(core_axis_name="c", subcore_axis_name="s")


@functools.partial(
    pl.kernel,
    mesh=_mesh,
    out_type=jax.ShapeDtypeStruct((2, NPAD, H), jnp.float32),
    scratch_types=[
        pltpu.VMEM((NBLK, K), jnp.int32),
        pltpu.VMEM((K, H), jnp.float32),
        pltpu.VMEM_SHARED((NPAD, H), jnp.float32),
    ],
)
def _deg_sc(dst_hbm, out_hbm, didx, ones_v, acc):
    c = lax.axis_index("c")
    s = lax.axis_index("s")
    wid = s * 2 + c

    def fill(val):
        def row(i, carry):
            for j in range(H // 16):
                ones_v[i, pl.ds(j * 16, 16)] = jnp.full((16,), val, jnp.float32)
            return carry
        lax.fori_loop(0, K, row, None)

    fill(0.0)
    for j in range(RPS // K):
        pltpu.sync_copy(ones_v, acc.at[pl.ds(s * RPS + j * K, K)])
    pltpu.sync_copy(dst_hbm.at[wid], didx)
    fill(1.0)
    plsc.subcore_barrier()

    def body(b, carry):
        pltpu.sync_copy(ones_v, acc.at[didx.at[b]], add=True)
        return carry

    lax.fori_loop(0, NBLK, body, None)
    plsc.subcore_barrier()
    pltpu.sync_copy(acc.at[pl.ds(s * RPS, RPS)], out_hbm.at[c, pl.ds(s * RPS, RPS)])


@functools.partial(
    pl.kernel,
    mesh=_mesh,
    out_type=jax.ShapeDtypeStruct((2, NPAD, H), jnp.float32),
    scratch_types=[
        pltpu.VMEM((NBLK, K), jnp.int32),
        pltpu.VMEM((NBLK, K), jnp.int32),
        pltpu.VMEM((K, H), jnp.float32),
        pltpu.VMEM_SHARED((NPAD, H), jnp.float32),
        pltpu.SemaphoreType.DMA,
    ],
)
def _hop_sc(table_hbm, src_hbm, dst_hbm, out_hbm, sidx, didx, vals, acc, sem):
    c = lax.axis_index("c")
    s = lax.axis_index("s")
    wid = s * 2 + c

    def zrow(i, carry):
        for j in range(H // 16):
            vals[i, pl.ds(j * 16, 16)] = jnp.zeros((16,), jnp.float32)
        return carry

    lax.fori_loop(0, K, zrow, None)
    for j in range(RPS // K):
        pltpu.sync_copy(vals, acc.at[pl.ds(s * RPS + j * K, K)])
    pltpu.sync_copy(src_hbm.at[wid], sidx)
    pltpu.sync_copy(dst_hbm.at[wid], didx)
    plsc.subcore_barrier()

    def body(b, carry):
        pltpu.async_copy(table_hbm.at[sidx.at[b]], vals, sem).wait()
        pltpu.sync_copy(vals, acc.at[didx.at[b]], add=True)
        return carry

    lax.fori_loop(0, NBLK, body, None)
    plsc.subcore_barrier()
    pltpu.sync_copy(acc.at[pl.ds(s * RPS, RPS)], out_hbm.at[c, pl.ds(s * RPS, RPS)])


def _r_of(d_ref):
    deg = d_ref[0][:, 0:1] + d_ref[1][:, 0:1]
    return lax.rsqrt(jnp.maximum(deg, 1.0))


def _init_body(x_ref, w_ref, b_ref, d_ref, g_ref):
    h = jnp.dot(x_ref[...], w_ref[...], preferred_element_type=jnp.float32)
    g_ref[...] = (h + b_ref[...]) * _r_of(d_ref)


def _mid_body(pa_ref, pb_ref, d_ref, w_ref, b_ref, out0_ref, g2_ref):
    r = _r_of(d_ref)
    cur = (pa_ref[...] + pb_ref[...]) * r
    out0_ref[...] = (
        jnp.dot(cur, w_ref[...], preferred_element_type=jnp.float32) + b_ref[...]
    )
    g2_ref[...] = cur * r


def _fin_body(pa_ref, pb_ref, d_ref, w_ref, b_ref, o0_ref, lg_ref, lb_ref,
              wpt_ref, bp_ref, out_ref):
    r = _r_of(d_ref)
    cur = (pa_ref[...] + pb_ref[...]) * r
    o1 = jnp.dot(cur, w_ref[...], preferred_element_type=jnp.float32) + b_ref[...]
    hm = 0.5 * (o0_ref[...] + o1)
    mu = jnp.mean(hm, axis=1, keepdims=True)
    dc = hm - mu
    var = jnp.mean(dc * dc, axis=1, keepdims=True)
    hn = dc * lax.rsqrt(var + 1e-5) * lg_ref[...] + lb_ref[...]
    hn = jnp.maximum(hn, 0.0)
    logit = jnp.sum(hn * wpt_ref[...], axis=1, keepdims=True) + bp_ref[...]
    out_ref[...] = jax.nn.sigmoid(logit)


def _row_spec():
    return pl.BlockSpec((BR, H), lambda i: (i, 0))


def _full_spec(shape):
    nd = len(shape)
    return pl.BlockSpec(shape, lambda i: (0,) * nd)


def _deg_spec():
    return pl.BlockSpec((2, BR, H), lambda i: (0, i, 0))


def kernel(x, edge_index, W_init, b_init, W_ord, b_ord, ln_g, ln_b, Wp, bp):
    src2d = edge_index[0].reshape(NW, NBLK, K)
    dst2d = edge_index[1].reshape(NW, NBLK, K)

    deg_p = _deg_sc(dst2d)

    g1 = pl.pallas_call(
        _init_body,
        grid=(GRID,),
        in_specs=[_row_spec(), _full_spec((H, H)), _full_spec((1, H)), _deg_spec()],
        out_specs=_row_spec(),
        out_shape=jax.ShapeDtypeStruct((N, H), jnp.float32),
    )(x, W_init, b_init.reshape(1, H), deg_p)

    p1 = _hop_sc(g1, src2d, dst2d)

    out0, g2 = pl.pallas_call(
        _mid_body,
        grid=(GRID,),
        in_specs=[_row_spec(), _row_spec(), _deg_spec(),
                  _full_spec((H, H)), _full_spec((1, H))],
        out_specs=[_row_spec(), _row_spec()],
        out_shape=[jax.ShapeDtypeStruct((N, H), jnp.float32),
                   jax.ShapeDtypeStruct((N, H), jnp.float32)],
    )(p1[0, :N], p1[1, :N], deg_p, W_ord[0], b_ord[0].reshape(1, H))

    p2 = _hop_sc(g2, src2d, dst2d)

    pred = pl.pallas_call(
        _fin_body,
        grid=(GRID,),
        in_specs=[_row_spec(), _row_spec(), _deg_spec(),
                  _full_spec((H, H)), _full_spec((1, H)), _row_spec(),
                  _full_spec((1, H)), _full_spec((1, H)),
                  _full_spec((1, H)), _full_spec((1, 1))],
        out_specs=pl.BlockSpec((BR, 1), lambda i: (i, 0)),
        out_shape=jax.ShapeDtypeStruct((N, 1), jnp.float32),
    )(p2[0, :N], p2[1, :N], deg_p, W_ord[1], b_ord[1].reshape(1, H), out0,
      ln_g.reshape(1, H), ln_b.reshape(1, H), Wp.reshape(1, H),
      bp.reshape(1, 1))

    return pred.reshape(1, -1)


# speedup vs baseline: 21.1945x; 1.4547x over previous
import functools

import jax
import jax.numpy as jnp
from jax import lax
from jax.experimental import pallas as pl
from jax.experimental.pallas import tpu as pltpu, tpu_sc as plsc

N = 10000
E = 320000
H = 128

NW = 32
K = 80
EPW = E // NW
NBLK = EPW // K
NPAD = 10240
RPS = NPAD // 16
BR = 1024
GRID = NPAD // BR

_mesh = plsc.VectorSubcoreMesh(core_axis_name="c", subcore_axis_name="s")


@functools.partial(
    pl.kernel,
    mesh=_mesh,
    out_type=jax.ShapeDtypeStruct((2, NPAD, H), jnp.float32),
    scratch_types=[
        pltpu.VMEM((NBLK, K), jnp.int32),
        pltpu.VMEM((K, H), jnp.float32),
        pltpu.VMEM_SHARED((NPAD, H), jnp.float32),
    ],
)
def _deg_sc(dst_hbm, out_hbm, didx, ones_v, acc):
    c = lax.axis_index("c")
    s = lax.axis_index("s")
    wid = s * 2 + c

    def fill(val):
        def row(i, carry):
            for j in range(H // 16):
                ones_v[i, pl.ds(j * 16, 16)] = jnp.full((16,), val, jnp.float32)
            return carry
        lax.fori_loop(0, K, row, None)

    fill(0.0)
    for j in range(RPS // K):
        pltpu.sync_copy(ones_v, acc.at[pl.ds(s * RPS + j * K, K)])
    pltpu.sync_copy(dst_hbm.at[wid], didx)
    fill(1.0)
    plsc.subcore_barrier()

    def body(b, carry):
        pltpu.sync_copy(ones_v, acc.at[didx.at[b]], add=True)
        return carry

    lax.fori_loop(0, NBLK, body, None)
    plsc.subcore_barrier()
    pltpu.sync_copy(acc.at[pl.ds(s * RPS, RPS)], out_hbm.at[c, pl.ds(s * RPS, RPS)])


@functools.partial(
    pl.kernel,
    mesh=_mesh,
    out_type=jax.ShapeDtypeStruct((2, NPAD, H), jnp.float32),
    scratch_types=[
        pltpu.VMEM((NBLK, K), jnp.int32),
        pltpu.VMEM((2, 1, K), jnp.int32),
        pltpu.VMEM((2, K, H), jnp.float32),
        pltpu.VMEM_SHARED((NPAD, H), jnp.float32),
        pltpu.SemaphoreType.DMA((2,)),
        pltpu.SemaphoreType.DMA((2,)),
    ],
)
def _hop_sc(table_hbm, src_hbm, dst_hbm, out_hbm, sidx, didxb, vals, acc,
            gsems, dsems):
    c = lax.axis_index("c")
    s = lax.axis_index("s")
    wid = s * 2 + c

    def zrow(i, carry):
        for j in range(H // 16):
            vals[0, i, pl.ds(j * 16, 16)] = jnp.zeros((16,), jnp.float32)
        return carry

    lax.fori_loop(0, K, zrow, None)
    for j in range(RPS // K):
        pltpu.sync_copy(vals.at[0], acc.at[pl.ds(s * RPS + j * K, K)])
    pltpu.sync_copy(src_hbm.at[wid], sidx)
    plsc.subcore_barrier()

    pltpu.async_copy(table_hbm.at[sidx.at[0]], vals.at[0], gsems.at[0])
    pltpu.async_copy(dst_hbm.at[wid, pl.ds(0, 1)], didxb.at[0], dsems.at[0])

    def body(b, carry):
        slot = lax.rem(b, 2)
        nslot = lax.rem(b + 1, 2)

        @pl.when(b + 1 < NBLK)
        def _():
            pltpu.async_copy(table_hbm.at[sidx.at[b + 1]], vals.at[nslot],
                             gsems.at[nslot])
            pltpu.async_copy(dst_hbm.at[wid, pl.ds(b + 1, 1)], didxb.at[nslot],
                             dsems.at[nslot])

        pltpu.make_async_copy(table_hbm.at[sidx.at[b]], vals.at[slot],
                              gsems.at[slot]).wait()
        pltpu.make_async_copy(dst_hbm.at[wid, pl.ds(b, 1)], didxb.at[slot],
                              dsems.at[slot]).wait()
        pltpu.sync_copy(vals.at[slot], acc.at[didxb.at[slot, 0]], add=True)
        return carry

    lax.fori_loop(0, NBLK, body, None)
    plsc.subcore_barrier()
    pltpu.sync_copy(acc.at[pl.ds(s * RPS, RPS)], out_hbm.at[c, pl.ds(s * RPS, RPS)])


def _r_of(d_ref):
    deg = d_ref[0][:, 0:1] + d_ref[1][:, 0:1]
    return lax.rsqrt(jnp.maximum(deg, 1.0))


def _init_body(x_ref, w_ref, b_ref, d_ref, g_ref):
    h = jnp.dot(x_ref[...], w_ref[...], preferred_element_type=jnp.float32)
    g_ref[...] = (h + b_ref[...]) * _r_of(d_ref)


def _mid_body(pa_ref, pb_ref, d_ref, w_ref, b_ref, out0_ref, g2_ref):
    r = _r_of(d_ref)
    cur = (pa_ref[...] + pb_ref[...]) * r
    out0_ref[...] = (
        jnp.dot(cur, w_ref[...], preferred_element_type=jnp.float32) + b_ref[...]
    )
    g2_ref[...] = cur * r


def _fin_body(pa_ref, pb_ref, d_ref, w_ref, b_ref, o0_ref, lg_ref, lb_ref,
              wpt_ref, bp_ref, out_ref):
    r = _r_of(d_ref)
    cur = (pa_ref[...] + pb_ref[...]) * r
    o1 = jnp.dot(cur, w_ref[...], preferred_element_type=jnp.float32) + b_ref[...]
    hm = 0.5 * (o0_ref[...] + o1)
    mu = jnp.mean(hm, axis=1, keepdims=True)
    dc = hm - mu
    var = jnp.mean(dc * dc, axis=1, keepdims=True)
    hn = dc * lax.rsqrt(var + 1e-5) * lg_ref[...] + lb_ref[...]
    hn = jnp.maximum(hn, 0.0)
    logit = jnp.sum(hn * wpt_ref[...], axis=1, keepdims=True) + bp_ref[...]
    out_ref[...] = jax.nn.sigmoid(logit)


def _row_spec():
    return pl.BlockSpec((BR, H), lambda i: (i, 0))


def _full_spec(shape):
    nd = len(shape)
    return pl.BlockSpec(shape, lambda i: (0,) * nd)


def _deg_spec():
    return pl.BlockSpec((2, BR, H), lambda i: (0, i, 0))


def kernel(x, edge_index, W_init, b_init, W_ord, b_ord, ln_g, ln_b, Wp, bp):
    src2d = edge_index[0].reshape(NW, NBLK, K)
    dst2d = edge_index[1].reshape(NW, NBLK, K)

    deg_p = _deg_sc(dst2d)

    g1 = pl.pallas_call(
        _init_body,
        grid=(GRID,),
        in_specs=[_row_spec(), _full_spec((H, H)), _full_spec((1, H)), _deg_spec()],
        out_specs=_row_spec(),
        out_shape=jax.ShapeDtypeStruct((N, H), jnp.float32),
    )(x, W_init, b_init.reshape(1, H), deg_p)

    p1 = _hop_sc(g1, src2d, dst2d)

    out0, g2 = pl.pallas_call(
        _mid_body,
        grid=(GRID,),
        in_specs=[_row_spec(), _row_spec(), _deg_spec(),
                  _full_spec((H, H)), _full_spec((1, H))],
        out_specs=[_row_spec(), _row_spec()],
        out_shape=[jax.ShapeDtypeStruct((N, H), jnp.float32),
                   jax.ShapeDtypeStruct((N, H), jnp.float32)],
    )(p1[0, :N], p1[1, :N], deg_p, W_ord[0], b_ord[0].reshape(1, H))

    p2 = _hop_sc(g2, src2d, dst2d)

    pred = pl.pallas_call(
        _fin_body,
        grid=(GRID,),
        in_specs=[_row_spec(), _row_spec(), _deg_spec(),
                  _full_spec((H, H)), _full_spec((1, H)), _row_spec(),
                  _full_spec((1, H)), _full_spec((1, H)),
                  _full_spec((1, H)), _full_spec((1, 1))],
        out_specs=pl.BlockSpec((BR, 1), lambda i: (i, 0)),
        out_shape=jax.ShapeDtypeStruct((N, 1), jnp.float32),
    )(p2[0, :N], p2[1, :N], deg_p, W_ord[1], b_ord[1].reshape(1, H), out0,
      ln_g.reshape(1, H), ln_b.reshape(1, H), Wp.reshape(1, H),
      bp.reshape(1, 1))

    return pred.reshape(1, -1)


# speedup vs baseline: 25.0139x; 1.1802x over previous
import functools

import jax
import jax.numpy as jnp
from jax import lax
from jax.experimental import pallas as pl
from jax.experimental.pallas import tpu as pltpu, tpu_sc as plsc

N = 10000
E = 320000
H = 128

NW = 32
K = 80
EPW = E // NW
NBLK = EPW // K
NPAD = 10240
RPS = NPAD // 16
BR = 1024
GRID = NPAD // BR

_mesh = plsc.VectorSubcoreMesh(core_axis_name="c", subcore_axis_name="s")


@functools.partial(
    pl.kernel,
    mesh=_mesh,
    out_type=jax.ShapeDtypeStruct((2, NPAD), jnp.float32),
    scratch_types=[
        pltpu.VMEM((NBLK, K), jnp.int32),
        pltpu.VMEM((K,), jnp.float32),
        pltpu.VMEM((RPS,), jnp.float32),
        pltpu.VMEM_SHARED((NPAD,), jnp.float32),
    ],
)
def _deg_sc(dst_hbm, out_hbm, didx, ones_v, zbuf, acc):
    c = lax.axis_index("c")
    s = lax.axis_index("s")
    wid = s * 2 + c

    def zrow(i, carry):
        zbuf[pl.ds(i * 16, 16)] = jnp.zeros((16,), jnp.float32)
        return carry

    lax.fori_loop(0, RPS // 16, zrow, None)

    def orow(i, carry):
        ones_v[pl.ds(i * 16, 16)] = jnp.full((16,), 1.0, jnp.float32)
        return carry

    lax.fori_loop(0, K // 16, orow, None)

    pltpu.sync_copy(zbuf, acc.at[pl.ds(s * RPS, RPS)])
    pltpu.sync_copy(dst_hbm.at[wid], didx)
    plsc.subcore_barrier()

    def body(b, carry):
        pltpu.sync_copy(ones_v, acc.at[didx.at[b]], add=True)
        return carry

    lax.fori_loop(0, NBLK, body, None)
    plsc.subcore_barrier()
    pltpu.sync_copy(acc.at[pl.ds(s * RPS, RPS)], out_hbm.at[c, pl.ds(s * RPS, RPS)])


@functools.partial(
    pl.kernel,
    mesh=_mesh,
    out_type=jax.ShapeDtypeStruct((2, NPAD, H), jnp.float32),
    scratch_types=[
        pltpu.VMEM((NBLK, K), jnp.int32),
        pltpu.VMEM((2, 1, K), jnp.int32),
        pltpu.VMEM((2, K, H), jnp.float32),
        pltpu.VMEM_SHARED((NPAD, H), jnp.float32),
        pltpu.SemaphoreType.DMA((2,)),
        pltpu.SemaphoreType.DMA((2,)),
    ],
)
def _hop_sc(table_hbm, src_hbm, dst_hbm, out_hbm, sidx, didxb, vals, acc,
            gsems, dsems):
    c = lax.axis_index("c")
    s = lax.axis_index("s")
    wid = s * 2 + c

    def zrow(i, carry):
        for j in range(H // 16):
            vals[0, i, pl.ds(j * 16, 16)] = jnp.zeros((16,), jnp.float32)
        return carry

    lax.fori_loop(0, K, zrow, None)
    for j in range(RPS // K):
        pltpu.sync_copy(vals.at[0], acc.at[pl.ds(s * RPS + j * K, K)])
    pltpu.sync_copy(src_hbm.at[wid], sidx)
    plsc.subcore_barrier()

    pltpu.async_copy(table_hbm.at[sidx.at[0]], vals.at[0], gsems.at[0])
    pltpu.async_copy(dst_hbm.at[wid, pl.ds(0, 1)], didxb.at[0], dsems.at[0])

    def body(b, carry):
        slot = lax.rem(b, 2)
        nslot = lax.rem(b + 1, 2)

        @pl.when(b + 1 < NBLK)
        def _():
            pltpu.async_copy(table_hbm.at[sidx.at[b + 1]], vals.at[nslot],
                             gsems.at[nslot])
            pltpu.async_copy(dst_hbm.at[wid, pl.ds(b + 1, 1)], didxb.at[nslot],
                             dsems.at[nslot])

        pltpu.make_async_copy(table_hbm.at[sidx.at[b]], vals.at[slot],
                              gsems.at[slot]).wait()
        pltpu.make_async_copy(dst_hbm.at[wid, pl.ds(b, 1)], didxb.at[slot],
                              dsems.at[slot]).wait()
        pltpu.sync_copy(vals.at[slot], acc.at[didxb.at[slot, 0]], add=True)
        return carry

    lax.fori_loop(0, NBLK, body, None)
    plsc.subcore_barrier()
    pltpu.sync_copy(acc.at[pl.ds(s * RPS, RPS)], out_hbm.at[c, pl.ds(s * RPS, RPS)])


def _r_of(d_ref):
    deg = d_ref[0] + d_ref[1]
    return lax.rsqrt(jnp.maximum(deg, 1.0))


def _init_body(x_ref, w_ref, b_ref, d_ref, g_ref):
    h = jnp.dot(x_ref[...], w_ref[...], preferred_element_type=jnp.float32)
    g_ref[...] = (h + b_ref[...]) * _r_of(d_ref)


def _mid_body(p_ref, d_ref, w_ref, b_ref, out0_ref, g2_ref):
    r = _r_of(d_ref)
    cur = (p_ref[0] + p_ref[1]) * r
    out0_ref[...] = (
        jnp.dot(cur, w_ref[...], preferred_element_type=jnp.float32) + b_ref[...]
    )
    g2_ref[...] = cur * r


def _fin_body(p_ref, d_ref, w_ref, b_ref, o0_ref, lg_ref, lb_ref,
              wpt_ref, bp_ref, out_ref):
    r = _r_of(d_ref)
    cur = (p_ref[0] + p_ref[1]) * r
    o1 = jnp.dot(cur, w_ref[...], preferred_element_type=jnp.float32) + b_ref[...]
    hm = 0.5 * (o0_ref[...] + o1)
    mu = jnp.mean(hm, axis=1, keepdims=True)
    dc = hm - mu
    var = jnp.mean(dc * dc, axis=1, keepdims=True)
    hn = dc * lax.rsqrt(var + 1e-5) * lg_ref[...] + lb_ref[...]
    hn = jnp.maximum(hn, 0.0)
    logit = jnp.sum(hn * wpt_ref[...], axis=1, keepdims=True) + bp_ref[...]
    out_ref[...] = jax.nn.sigmoid(logit)


def _row_spec():
    return pl.BlockSpec((BR, H), lambda i: (i, 0))


def _full_spec(shape):
    nd = len(shape)
    return pl.BlockSpec(shape, lambda i: (0,) * nd)


def _deg_spec():
    return pl.BlockSpec((2, BR, 1), lambda i: (0, i, 0))


def _pair_spec():
    return pl.BlockSpec((2, BR, H), lambda i: (0, i, 0))


def kernel(x, edge_index, W_init, b_init, W_ord, b_ord, ln_g, ln_b, Wp, bp):
    src2d = edge_index[0].reshape(NW, NBLK, K)
    dst2d = edge_index[1].reshape(NW, NBLK, K)

    deg_p = _deg_sc(dst2d).reshape(2, NPAD, 1)

    g1 = pl.pallas_call(
        _init_body,
        grid=(GRID,),
        in_specs=[_row_spec(), _full_spec((H, H)), _full_spec((1, H)), _deg_spec()],
        out_specs=_row_spec(),
        out_shape=jax.ShapeDtypeStruct((N, H), jnp.float32),
    )(x, W_init, b_init.reshape(1, H), deg_p)

    p1 = _hop_sc(g1, src2d, dst2d)

    out0, g2 = pl.pallas_call(
        _mid_body,
        grid=(GRID,),
        in_specs=[_pair_spec(), _deg_spec(),
                  _full_spec((H, H)), _full_spec((1, H))],
        out_specs=[_row_spec(), _row_spec()],
        out_shape=[jax.ShapeDtypeStruct((N, H), jnp.float32),
                   jax.ShapeDtypeStruct((N, H), jnp.float32)],
    )(p1, deg_p, W_ord[0], b_ord[0].reshape(1, H))

    p2 = _hop_sc(g2, src2d, dst2d)

    pred = pl.pallas_call(
        _fin_body,
        grid=(GRID,),
        in_specs=[_pair_spec(), _deg_spec(),
                  _full_spec((H, H)), _full_spec((1, H)), _row_spec(),
                  _full_spec((1, H)), _full_spec((1, H)),
                  _full_spec((1, H)), _full_spec((1, 1))],
        out_specs=pl.BlockSpec((BR, 1), lambda i: (i, 0)),
        out_shape=jax.ShapeDtypeStruct((N, 1), jnp.float32),
    )(p2, deg_p, W_ord[1], b_ord[1].reshape(1, H), out0,
      ln_g.reshape(1, H), ln_b.reshape(1, H), Wp.reshape(1, H),
      bp.reshape(1, 1))

    return pred.reshape(1, -1)


# speedup vs baseline: 25.3892x; 1.0150x over previous
import functools

import jax
import jax.numpy as jnp
from jax import lax
from jax.experimental import pallas as pl
from jax.experimental.pallas import tpu as pltpu, tpu_sc as plsc

N = 10000
E = 320000
H = 128

NW = 32
K = 80
EPW = E // NW
NBLK = EPW // K
NPAD = 10240
RPS = NPAD // 16
BR = 1024
GRID = NPAD // BR

_mesh = plsc.VectorSubcoreMesh(core_axis_name="c", subcore_axis_name="s")


@functools.partial(
    pl.kernel,
    mesh=_mesh,
    out_type=jax.ShapeDtypeStruct((2, NPAD), jnp.float32),
    scratch_types=[
        pltpu.VMEM((NBLK, K), jnp.int32),
        pltpu.VMEM((K,), jnp.float32),
        pltpu.VMEM((RPS,), jnp.float32),
        pltpu.VMEM_SHARED((NPAD,), jnp.float32),
        pltpu.SemaphoreType.DMA((2,)),
    ],
)
def _deg_sc(dst_hbm, out_hbm, didx, ones_v, zbuf, acc, ssems):
    c = lax.axis_index("c")
    s = lax.axis_index("s")
    wid = s * 2 + c

    def zrow(i, carry):
        zbuf[pl.ds(i * 16, 16)] = jnp.zeros((16,), jnp.float32)
        return carry

    lax.fori_loop(0, RPS // 16, zrow, None)

    def orow(i, carry):
        ones_v[pl.ds(i * 16, 16)] = jnp.full((16,), 1.0, jnp.float32)
        return carry

    lax.fori_loop(0, K // 16, orow, None)

    pltpu.sync_copy(zbuf, acc.at[pl.ds(s * RPS, RPS)])
    pltpu.sync_copy(dst_hbm.at[wid], didx)
    plsc.subcore_barrier()

    def body(b, carry):
        slot = lax.rem(b, 2)

        @pl.when(b >= 2)
        def _():
            pltpu.make_async_copy(ones_v, acc.at[didx.at[b - 2]],
                                  ssems.at[slot]).wait()

        pltpu.async_copy(ones_v, acc.at[didx.at[b]], ssems.at[slot], add=True)
        return carry

    lax.fori_loop(0, NBLK, body, None)
    for t in range(2):
        pltpu.make_async_copy(ones_v, acc.at[didx.at[NBLK - 2 + t]],
                              ssems.at[(NBLK - 2 + t) % 2]).wait()
    plsc.subcore_barrier()
    pltpu.sync_copy(acc.at[pl.ds(s * RPS, RPS)], out_hbm.at[c, pl.ds(s * RPS, RPS)])


@functools.partial(
    pl.kernel,
    mesh=_mesh,
    out_type=jax.ShapeDtypeStruct((2, NPAD, H), jnp.float32),
    scratch_types=[
        pltpu.VMEM((NBLK, K), jnp.int32),
        pltpu.VMEM((2, 1, K), jnp.int32),
        pltpu.VMEM((2, K, H), jnp.float32),
        pltpu.VMEM_SHARED((NPAD, H), jnp.float32),
        pltpu.SemaphoreType.DMA((2,)),
        pltpu.SemaphoreType.DMA((2,)),
        pltpu.SemaphoreType.DMA((2,)),
    ],
)
def _hop_sc(table_hbm, src_hbm, dst_hbm, out_hbm, sidx, didxb, vals, acc,
            gsems, dsems, ssems):
    c = lax.axis_index("c")
    s = lax.axis_index("s")
    wid = s * 2 + c

    def zrow(i, carry):
        for j in range(H // 16):
            vals[0, i, pl.ds(j * 16, 16)] = jnp.zeros((16,), jnp.float32)
        return carry

    lax.fori_loop(0, K, zrow, None)
    for j in range(RPS // K):
        pltpu.sync_copy(vals.at[0], acc.at[pl.ds(s * RPS + j * K, K)])
    pltpu.sync_copy(src_hbm.at[wid], sidx)
    plsc.subcore_barrier()

    pltpu.async_copy(table_hbm.at[sidx.at[0]], vals.at[0], gsems.at[0])
    pltpu.async_copy(dst_hbm.at[wid, pl.ds(0, 1)], didxb.at[0], dsems.at[0])

    def body(b, carry):
        slot = lax.rem(b, 2)
        nslot = lax.rem(b + 1, 2)

        @pl.when(b >= 1)
        def _():
            pltpu.make_async_copy(vals.at[nslot], acc.at[didxb.at[nslot, 0]],
                                  ssems.at[nslot]).wait()

        @pl.when(b + 1 < NBLK)
        def _():
            pltpu.async_copy(table_hbm.at[sidx.at[b + 1]], vals.at[nslot],
                             gsems.at[nslot])
            pltpu.async_copy(dst_hbm.at[wid, pl.ds(b + 1, 1)], didxb.at[nslot],
                             dsems.at[nslot])

        pltpu.make_async_copy(table_hbm.at[sidx.at[b]], vals.at[slot],
                              gsems.at[slot]).wait()
        pltpu.make_async_copy(dst_hbm.at[wid, pl.ds(b, 1)], didxb.at[slot],
                              dsems.at[slot]).wait()
        pltpu.async_copy(vals.at[slot], acc.at[didxb.at[slot, 0]],
                         ssems.at[slot], add=True)
        return carry

    lax.fori_loop(0, NBLK, body, None)
    last = (NBLK - 1) % 2
    pltpu.make_async_copy(vals.at[last], acc.at[didxb.at[last, 0]],
                          ssems.at[last]).wait()
    plsc.subcore_barrier()
    pltpu.sync_copy(acc.at[pl.ds(s * RPS, RPS)], out_hbm.at[c, pl.ds(s * RPS, RPS)])


def _r_of(d_ref):
    deg = d_ref[0] + d_ref[1]
    return lax.rsqrt(jnp.maximum(deg, 1.0))


def _init_body(x_ref, w_ref, b_ref, d_ref, g_ref):
    h = jnp.dot(x_ref[...], w_ref[...], preferred_element_type=jnp.float32)
    g_ref[...] = (h + b_ref[...]) * _r_of(d_ref)


def _mid_body(p_ref, d_ref, w_ref, b_ref, out0_ref, g2_ref):
    r = _r_of(d_ref)
    cur = (p_ref[0] + p_ref[1]) * r
    out0_ref[...] = (
        jnp.dot(cur, w_ref[...], preferred_element_type=jnp.float32) + b_ref[...]
    )
    g2_ref[...] = cur * r


def _fin_body(p_ref, d_ref, w_ref, b_ref, o0_ref, lg_ref, lb_ref,
              wpt_ref, bp_ref, out_ref):
    r = _r_of(d_ref)
    cur = (p_ref[0] + p_ref[1]) * r
    o1 = jnp.dot(cur, w_ref[...], preferred_element_type=jnp.float32) + b_ref[...]
    hm = 0.5 * (o0_ref[...] + o1)
    mu = jnp.mean(hm, axis=1, keepdims=True)
    dc = hm - mu
    var = jnp.mean(dc * dc, axis=1, keepdims=True)
    hn = dc * lax.rsqrt(var + 1e-5) * lg_ref[...] + lb_ref[...]
    hn = jnp.maximum(hn, 0.0)
    logit = jnp.sum(hn * wpt_ref[...], axis=1, keepdims=True) + bp_ref[...]
    out_ref[...] = jax.nn.sigmoid(logit)


def _row_spec():
    return pl.BlockSpec((BR, H), lambda i: (i, 0))


def _full_spec(shape):
    nd = len(shape)
    return pl.BlockSpec(shape, lambda i: (0,) * nd)


def _deg_spec():
    return pl.BlockSpec((2, BR, 1), lambda i: (0, i, 0))


def _pair_spec():
    return pl.BlockSpec((2, BR, H), lambda i: (0, i, 0))


def kernel(x, edge_index, W_init, b_init, W_ord, b_ord, ln_g, ln_b, Wp, bp):
    src2d = edge_index[0].reshape(NW, NBLK, K)
    dst2d = edge_index[1].reshape(NW, NBLK, K)

    deg_p = _deg_sc(dst2d).reshape(2, NPAD, 1)

    g1 = pl.pallas_call(
        _init_body,
        grid=(GRID,),
        in_specs=[_row_spec(), _full_spec((H, H)), _full_spec((1, H)), _deg_spec()],
        out_specs=_row_spec(),
        out_shape=jax.ShapeDtypeStruct((N, H), jnp.float32),
    )(x, W_init, b_init.reshape(1, H), deg_p)

    p1 = _hop_sc(g1, src2d, dst2d)

    out0, g2 = pl.pallas_call(
        _mid_body,
        grid=(GRID,),
        in_specs=[_pair_spec(), _deg_spec(),
                  _full_spec((H, H)), _full_spec((1, H))],
        out_specs=[_row_spec(), _row_spec()],
        out_shape=[jax.ShapeDtypeStruct((N, H), jnp.float32),
                   jax.ShapeDtypeStruct((N, H), jnp.float32)],
    )(p1, deg_p, W_ord[0], b_ord[0].reshape(1, H))

    p2 = _hop_sc(g2, src2d, dst2d)

    pred = pl.pallas_call(
        _fin_body,
        grid=(GRID,),
        in_specs=[_pair_spec(), _deg_spec(),
                  _full_spec((H, H)), _full_spec((1, H)), _row_spec(),
                  _full_spec((1, H)), _full_spec((1, H)),
                  _full_spec((1, H)), _full_spec((1, 1))],
        out_specs=pl.BlockSpec((BR, 1), lambda i: (i, 0)),
        out_shape=jax.ShapeDtypeStruct((N, 1), jnp.float32),
    )(p2, deg_p, W_ord[1], b_ord[1].reshape(1, H), out0,
      ln_g.reshape(1, H), ln_b.reshape(1, H), Wp.reshape(1, H),
      bp.reshape(1, 1))

    return pred.reshape(1, -1)
